# baseline (speedup 1.0000x reference)
import functools

import jax
import jax.numpy as jnp
from jax import lax
from jax.experimental import pallas as pl
from jax.experimental.pallas import tpu as pltpu
from jax.experimental.pallas import tpu_sc as plsc

EMBED_DIM = 128
BATCH = 16384
NUM_CORES = 2
NUM_SUBCORES = 16
NUM_WORKERS = NUM_CORES * NUM_SUBCORES
ROWS_PER_W = BATCH // NUM_WORKERS
CHUNK = 128
NCHUNK = ROWS_PER_W // CHUNK
LANES = 16
NVEC = EMBED_DIM // LANES


def _sc_encode(xt, emb_res, emb_inc, emb_typ, emb_wrk, W, b):
    mesh = plsc.VectorSubcoreMesh(core_axis_name="c", subcore_axis_name="s")

    @functools.partial(
        pl.kernel,
        mesh=mesh,
        out_type=jax.ShapeDtypeStruct((BATCH, EMBED_DIM), jnp.float32),
        scratch_types=[
            pltpu.VMEM((ROWS_PER_W,), jnp.int32),
            pltpu.VMEM((ROWS_PER_W,), jnp.int32),
            pltpu.VMEM((ROWS_PER_W,), jnp.int32),
            pltpu.VMEM((ROWS_PER_W,), jnp.int32),
            pltpu.VMEM((ROWS_PER_W,), jnp.int32),
            pltpu.VMEM((ROWS_PER_W,), jnp.int32),
            pltpu.VMEM((176,), jnp.float32),
            pltpu.VMEM((336,), jnp.float32),
            pltpu.VMEM((96,), jnp.float32),
            pltpu.VMEM((96,), jnp.float32),
            pltpu.VMEM((40, EMBED_DIM), jnp.float32),
            pltpu.VMEM((1, EMBED_DIM), jnp.float32),
            pltpu.VMEM((5, EMBED_DIM), jnp.float32),
            pltpu.VMEM((60, EMBED_DIM), jnp.float32),
            pltpu.VMEM((20, EMBED_DIM), jnp.float32),
            pltpu.VMEM((CHUNK, EMBED_DIM), jnp.float32),
            pltpu.VMEM((CHUNK, EMBED_DIM), jnp.float32),
            pltpu.VMEM((CHUNK, EMBED_DIM), jnp.float32),
            pltpu.VMEM((CHUNK, EMBED_DIM), jnp.float32),
            pltpu.VMEM_SHARED((60, EMBED_DIM), jnp.float32),
            pltpu.VMEM_SHARED((400, EMBED_DIM), jnp.float32),
            pltpu.VMEM_SHARED((100, EMBED_DIM), jnp.float32),
            pltpu.SemaphoreType.DMA,
            pltpu.SemaphoreType.DMA,
            pltpu.SemaphoreType.DMA,
            pltpu.SemaphoreType.DMA,
            pltpu.SemaphoreType.DMA,
        ],
    )
    def k(xth, resh, inch, typh, wrkh, wh, bh, outh,
          x0v, x1v, x2v, x3v, i12v, i34v,
          resv, incv, typv, wrkv, wv, bv, projv, pv, pairv,
          b12a, b12b, b34a, b34b, psh, t12s, t34s,
          sx, sga, sgb, soa, sob):
        b12s, b34s = [b12a, b12b], [b34a, b34b]
        sg, so = [sga, sgb], [soa, sob]
        sid = lax.axis_index("s")
        wid = sid * NUM_CORES + lax.axis_index("c")
        base = wid * ROWS_PER_W

        xcp = [pltpu.async_copy(xth.at[f, pl.ds(base, ROWS_PER_W)], xv, sx)
               for f, xv in enumerate([x0v, x1v, x2v, x3v])]
        pltpu.sync_copy(resh, resv.at[pl.ds(0, 160)])
        pltpu.sync_copy(inch, incv.at[pl.ds(0, 320)])
        pltpu.sync_copy(typh, typv.at[pl.ds(0, 80)])
        pltpu.sync_copy(wrkh, wrkv.at[pl.ds(0, 80)])
        pltpu.sync_copy(wh, wv)
        pltpu.sync_copy(bh, bv)

        def project(tbl, width, nrows_here, e0, woff, nterms, p0, add_bias):
            for l in range(nrows_here):
                e = e0 + l
                for j in range(NVEC):
                    sl = pl.ds(j * LANES, LANES)
                    if add_bias:
                        projv[l, sl] = bv[0, sl]
                    else:
                        projv[l, sl] = jnp.zeros((LANES,), jnp.float32)

                def terms(t, carry):
                    ev = tbl[pl.ds(e * width + t, LANES)]
                    s = ev[0]
                    for j in range(NVEC):
                        sl = pl.ds(j * LANES, LANES)
                        plsc.addupdate(projv.at[l, sl], s * wv[woff + t, sl])
                    return carry
                lax.fori_loop(0, nterms, terms, 0)
            pltpu.sync_copy(projv.at[pl.ds(0, nrows_here)],
                            psh.at[pl.ds(p0 + e0, nrows_here)])

        @pl.when(sid < 5)
        def _p1():
            project(resv, 8, 4, sid * 4, 0, 8, 0, False)

        @pl.when((sid >= 5) & (sid < 10))
        def _p2():
            project(incv, 16, 4, (sid - 5) * 4, 8, 16, 20, False)

        @pl.when((sid >= 10) & (sid < 12))
        def _p3():
            project(typv, 8, 5, (sid - 10) * 5, 24, 8, 40, False)

        @pl.when((sid >= 12) & (sid < 14))
        def _p4():
            project(wrkv, 8, 5, (sid - 12) * 5, 32, 8, 50, True)

        for c in xcp:
            c.wait()
        for r in range(ROWS_PER_W // LANES):
            sl = pl.ds(r * LANES, LANES)
            i12v[sl] = x0v[sl] * 20 + x1v[sl]
            i34v[sl] = x2v[sl] * 10 + x3v[sl]

        plsc.subcore_barrier()

        pltpu.sync_copy(psh, pv)

        def pair_rows(i_row, j0, jn, dst, dst_off):
            for l in range(jn):
                for j in range(NVEC):
                    sl = pl.ds(j * LANES, LANES)
                    pairv[l, sl] = pv[i_row, sl] + pv[j0 + l, sl]
            pltpu.sync_copy(pairv.at[pl.ds(0, jn)],
                            dst.at[pl.ds(dst_off, jn)])

        pair_rows(sid, 20, 20, t12s, sid * 20)

        @pl.when(sid < 4)
        def _t12b():
            i = sid + 16
            pair_rows(i, 20, 20, t12s, i * 20)

        @pl.when(sid >= 6)
        def _t34():
            i = sid - 6
            pair_rows(40 + i, 50, 10, t34s, i * 10)

        plsc.subcore_barrier()

        def issue(c):
            s = c % 2
            isl = pl.ds(c * CHUNK, CHUNK)
            return (pltpu.async_copy(t12s.at[i12v.at[isl]], b12s[s], sg[s]),
                    pltpu.async_copy(t34s.at[i34v.at[isl]], b34s[s], sg[s]))

        UNROLL = 4
        gcp = [None] * NCHUNK
        ocp = [None] * NCHUNK
        gcp[0] = issue(0)
        for c in range(NCHUNK):
            s = c % 2
            if c + 1 < NCHUNK:
                if c >= 1:
                    ocp[c - 1].wait()
                gcp[c + 1] = issue(c + 1)
            gcp[c][0].wait()
            gcp[c][1].wait()
            b12, b34 = b12s[s], b34s[s]

            def add_body(r, carry):
                for u in range(UNROLL):
                    for jj in range(NVEC):
                        sl = pl.ds(jj * LANES, LANES)
                        plsc.addupdate(b12.at[r * UNROLL + u, sl],
                                       b34[r * UNROLL + u, sl])
                return carry

            lax.fori_loop(0, CHUNK // UNROLL, add_body, 0)
            ocp[c] = pltpu.async_copy(
                b12, outh.at[pl.ds(base + c * CHUNK, CHUNK)], so[s])
        ocp[NCHUNK - 2].wait()
        ocp[NCHUNK - 1].wait()

    return k(xt, emb_res, emb_inc, emb_typ, emb_wrk, W, b)


def kernel(x, emb_res, emb_inc, emb_typ, emb_wrk, W, b):
    out = _sc_encode(x.astype(jnp.int32).T, emb_res.reshape(-1),
                     emb_inc.reshape(-1), emb_typ.reshape(-1),
                     emb_wrk.reshape(-1), W, b.reshape(1, EMBED_DIM))
    return out[:, None, :]


# speedup vs baseline: 1.3122x; 1.3122x over previous
import functools

import jax
import jax.numpy as jnp
from jax import lax
from jax.experimental import pallas as pl
from jax.experimental.pallas import tpu as pltpu
from jax.experimental.pallas import tpu_sc as plsc

EMBED_DIM = 128
BATCH = 16384
NUM_CORES = 2
NUM_SUBCORES = 16
NUM_WORKERS = NUM_CORES * NUM_SUBCORES
ROWS_PER_W = BATCH // NUM_WORKERS
CHUNK = 64
NCHUNK = ROWS_PER_W // CHUNK
RING = 3
LANES = 16


def _build_tables_body(res_ref, inc_ref, typ_ref, wrk_ref, w_ref, b_ref,
                       t12_ref, t34_ref):
    w = w_ref[...]
    t1 = jnp.dot(res_ref[...], w[0:8, :], preferred_element_type=jnp.float32)
    t2 = jnp.dot(inc_ref[...], w[8:24, :], preferred_element_type=jnp.float32)
    t3 = jnp.dot(typ_ref[...], w[24:32, :], preferred_element_type=jnp.float32)
    t4 = jnp.dot(wrk_ref[...], w[32:40, :], preferred_element_type=jnp.float32)
    bias = b_ref[...]
    for i in range(20):
        t12_ref[pl.ds(i * 20, 20), :] = t1[i:i + 1, :] + t2
    t4b = t4 + bias
    for i in range(10):
        t34_ref[pl.ds(i * 10, 10), :] = t3[i:i + 1, :] + t4b


def _build_tables(emb_res, emb_inc, emb_typ, emb_wrk, W, b):
    return pl.pallas_call(
        _build_tables_body,
        out_shape=(
            jax.ShapeDtypeStruct((400, EMBED_DIM), jnp.float32),
            jax.ShapeDtypeStruct((100, EMBED_DIM), jnp.float32),
        ),
    )(emb_res, emb_inc, emb_typ, emb_wrk, W, b.reshape(1, EMBED_DIM))


def _sc_lookup(xt, t12, t34):
    mesh = plsc.VectorSubcoreMesh(core_axis_name="c", subcore_axis_name="s")

    @functools.partial(
        pl.kernel,
        mesh=mesh,
        out_type=jax.ShapeDtypeStruct((BATCH, EMBED_DIM), jnp.float32),
        scratch_types=[
            pltpu.VMEM((ROWS_PER_W,), jnp.int32),
            pltpu.VMEM((ROWS_PER_W,), jnp.int32),
            pltpu.VMEM((ROWS_PER_W,), jnp.int32),
            pltpu.VMEM((ROWS_PER_W,), jnp.int32),
            pltpu.VMEM((ROWS_PER_W,), jnp.int32),
            pltpu.VMEM((ROWS_PER_W,), jnp.int32),
        ] + [pltpu.VMEM((CHUNK, EMBED_DIM), jnp.float32)
             for _ in range(RING)]
          + [pltpu.VMEM((CHUNK, EMBED_DIM), jnp.float32)
             for _ in range(RING)]
          + [
            pltpu.VMEM_SHARED((400, EMBED_DIM), jnp.float32),
            pltpu.VMEM_SHARED((100, EMBED_DIM), jnp.float32),
            pltpu.SemaphoreType.DMA,
        ] + [pltpu.SemaphoreType.DMA for _ in range(RING)]
          + [pltpu.SemaphoreType.DMA for _ in range(RING)],
    )
    def k(xth, t12h, t34h, outh,
          x0v, x1v, x2v, x3v, i12v, i34v, *rest):
        b12s = list(rest[0:RING])
        b34s = list(rest[RING:2 * RING])
        t12s, t34s, sx = rest[2 * RING], rest[2 * RING + 1], rest[2 * RING + 2]
        sg = list(rest[2 * RING + 3:2 * RING + 3 + RING])
        so = list(rest[2 * RING + 3 + RING:2 * RING + 3 + 2 * RING])
        sid = lax.axis_index("s")
        wid = sid * NUM_CORES + lax.axis_index("c")
        base = wid * ROWS_PER_W

        xcp = [pltpu.async_copy(xth.at[f, pl.ds(base, ROWS_PER_W)], xv, sx)
               for f, xv in enumerate([x0v, x1v, x2v, x3v])]

        @pl.when(sid == 0)
        def _stage_tables():
            pltpu.sync_copy(t12h, t12s)
            pltpu.sync_copy(t34h, t34s)

        for c in xcp:
            c.wait()
        for r in range(ROWS_PER_W // LANES):
            sl = pl.ds(r * LANES, LANES)
            i12v[sl] = x0v[sl] * 20 + x1v[sl]
            i34v[sl] = x2v[sl] * 10 + x3v[sl]

        plsc.subcore_barrier()

        def issue(c):
            s = c % RING
            isl = pl.ds(c * CHUNK, CHUNK)
            return (pltpu.async_copy(t12s.at[i12v.at[isl]], b12s[s], sg[s]),
                    pltpu.async_copy(t34s.at[i34v.at[isl]], b34s[s], sg[s]))

        UNROLL = 4
        LOOKAHEAD = RING - 1
        gcp = [None] * NCHUNK
        ocp = [None] * NCHUNK
        for c0 in range(min(LOOKAHEAD, NCHUNK)):
            gcp[c0] = issue(c0)
        for c in range(NCHUNK):
            s = c % RING
            n = c + LOOKAHEAD
            if n < NCHUNK:
                if n - RING >= 0:
                    ocp[n - RING].wait()
                gcp[n] = issue(n)
            gcp[c][0].wait()
            gcp[c][1].wait()
            b12, b34 = b12s[s], b34s[s]

            def add_body(r, carry):
                for u in range(UNROLL):
                    for jj in range(EMBED_DIM // LANES):
                        sl = pl.ds(jj * LANES, LANES)
                        plsc.addupdate(b12.at[r * UNROLL + u, sl],
                                       b34[r * UNROLL + u, sl])
                return carry

            lax.fori_loop(0, CHUNK // UNROLL, add_body, 0)
            ocp[c] = pltpu.async_copy(
                b12, outh.at[pl.ds(base + c * CHUNK, CHUNK)], so[s])
        for c in range(max(0, NCHUNK - RING), NCHUNK):
            ocp[c].wait()

    return k(xt, t12, t34)


def kernel(x, emb_res, emb_inc, emb_typ, emb_wrk, W, b):
    t12, t34 = _build_tables(emb_res, emb_inc, emb_typ, emb_wrk, W, b)
    out = _sc_lookup(x.astype(jnp.int32).T, t12, t34)
    return out[:, None, :]


# speedup vs baseline: 1.3283x; 1.0123x over previous
import functools

import jax
import jax.numpy as jnp
from jax import lax
from jax.experimental import pallas as pl
from jax.experimental.pallas import tpu as pltpu
from jax.experimental.pallas import tpu_sc as plsc

EMBED_DIM = 128
BATCH = 16384
NUM_CORES = 2
NUM_SUBCORES = 16
NUM_WORKERS = NUM_CORES * NUM_SUBCORES
ROWS_PER_W = BATCH // NUM_WORKERS
CHUNK = 64
NCHUNK = ROWS_PER_W // CHUNK
RING = 4
LANES = 16


def _build_tables_body(res_ref, inc_ref, typ_ref, wrk_ref, w_ref, b_ref,
                       t12_ref, t34_ref):
    w = w_ref[...]
    t1 = jnp.dot(res_ref[...], w[0:8, :], preferred_element_type=jnp.float32)
    t2 = jnp.dot(inc_ref[...], w[8:24, :], preferred_element_type=jnp.float32)
    t3 = jnp.dot(typ_ref[...], w[24:32, :], preferred_element_type=jnp.float32)
    t4 = jnp.dot(wrk_ref[...], w[32:40, :], preferred_element_type=jnp.float32)
    bias = b_ref[...]
    for i in range(20):
        t12_ref[pl.ds(i * 20, 20), :] = t1[i:i + 1, :] + t2
    t4b = t4 + bias
    for i in range(10):
        t34_ref[pl.ds(i * 10, 10), :] = t3[i:i + 1, :] + t4b


def _build_tables(emb_res, emb_inc, emb_typ, emb_wrk, W, b):
    return pl.pallas_call(
        _build_tables_body,
        out_shape=(
            jax.ShapeDtypeStruct((400, EMBED_DIM), jnp.float32),
            jax.ShapeDtypeStruct((100, EMBED_DIM), jnp.float32),
        ),
    )(emb_res, emb_inc, emb_typ, emb_wrk, W, b.reshape(1, EMBED_DIM))


def _sc_lookup(xt, t12, t34):
    mesh = plsc.VectorSubcoreMesh(core_axis_name="c", subcore_axis_name="s")

    @functools.partial(
        pl.kernel,
        mesh=mesh,
        out_type=jax.ShapeDtypeStruct((BATCH, EMBED_DIM), jnp.float32),
        scratch_types=[
            pltpu.VMEM((ROWS_PER_W,), jnp.int32),
            pltpu.VMEM((ROWS_PER_W,), jnp.int32),
            pltpu.VMEM((ROWS_PER_W,), jnp.int32),
            pltpu.VMEM((ROWS_PER_W,), jnp.int32),
            pltpu.VMEM((ROWS_PER_W,), jnp.int32),
            pltpu.VMEM((ROWS_PER_W,), jnp.int32),
        ] + [pltpu.VMEM((CHUNK, EMBED_DIM), jnp.float32)
             for _ in range(RING)]
          + [pltpu.VMEM((CHUNK, EMBED_DIM), jnp.float32)
             for _ in range(RING)]
          + [
            pltpu.VMEM_SHARED((400, EMBED_DIM), jnp.float32),
            pltpu.VMEM_SHARED((100, EMBED_DIM), jnp.float32),
            pltpu.SemaphoreType.DMA,
        ] + [pltpu.SemaphoreType.DMA for _ in range(RING)]
          + [pltpu.SemaphoreType.DMA for _ in range(RING)],
    )
    def k(xth, t12h, t34h, outh,
          x0v, x1v, x2v, x3v, i12v, i34v, *rest):
        b12s = list(rest[0:RING])
        b34s = list(rest[RING:2 * RING])
        t12s, t34s, sx = rest[2 * RING], rest[2 * RING + 1], rest[2 * RING + 2]
        sg = list(rest[2 * RING + 3:2 * RING + 3 + RING])
        so = list(rest[2 * RING + 3 + RING:2 * RING + 3 + 2 * RING])
        sid = lax.axis_index("s")
        wid = sid * NUM_CORES + lax.axis_index("c")
        base = wid * ROWS_PER_W

        xcp = [pltpu.async_copy(xth.at[f, pl.ds(base, ROWS_PER_W)], xv, sx)
               for f, xv in enumerate([x0v, x1v, x2v, x3v])]

        @pl.when(sid == 0)
        def _stage_tables():
            pltpu.sync_copy(t12h, t12s)
            pltpu.sync_copy(t34h, t34s)

        for c in xcp:
            c.wait()
        for r in range(ROWS_PER_W // LANES):
            sl = pl.ds(r * LANES, LANES)
            i12v[sl] = x0v[sl] * 20 + x1v[sl]
            i34v[sl] = x2v[sl] * 10 + x3v[sl]

        plsc.subcore_barrier()

        def issue(c):
            s = c % RING
            isl = pl.ds(c * CHUNK, CHUNK)
            return (pltpu.async_copy(t12s.at[i12v.at[isl]], b12s[s], sg[s]),
                    pltpu.async_copy(t34s.at[i34v.at[isl]], b34s[s], sg[s]))

        UNROLL = 4
        LOOKAHEAD = RING - 1
        gcp = [None] * NCHUNK
        ocp = [None] * NCHUNK
        for c0 in range(min(LOOKAHEAD, NCHUNK)):
            gcp[c0] = issue(c0)
        for c in range(NCHUNK):
            s = c % RING
            n = c + LOOKAHEAD
            if n < NCHUNK:
                if n - RING >= 0:
                    ocp[n - RING].wait()
                gcp[n] = issue(n)
            gcp[c][0].wait()
            gcp[c][1].wait()
            b12, b34 = b12s[s], b34s[s]

            def add_body(r, carry):
                for u in range(UNROLL):
                    for jj in range(EMBED_DIM // LANES):
                        sl = pl.ds(jj * LANES, LANES)
                        plsc.addupdate(b12.at[r * UNROLL + u, sl],
                                       b34[r * UNROLL + u, sl])
                return carry

            lax.fori_loop(0, CHUNK // UNROLL, add_body, 0)
            ocp[c] = pltpu.async_copy(
                b12, outh.at[pl.ds(base + c * CHUNK, CHUNK)], so[s])
        for c in range(max(0, NCHUNK - RING), NCHUNK):
            ocp[c].wait()

    return k(xt, t12, t34)


def kernel(x, emb_res, emb_inc, emb_typ, emb_wrk, W, b):
    t12, t34 = _build_tables(emb_res, emb_inc, emb_typ, emb_wrk, W, b)
    out = _sc_lookup(x.astype(jnp.int32).T, t12, t34)
    return out[:, None, :]


# speedup vs baseline: 1.3312x; 1.0022x over previous
import functools

import jax
import jax.numpy as jnp
from jax import lax
from jax.experimental import pallas as pl
from jax.experimental.pallas import tpu as pltpu
from jax.experimental.pallas import tpu_sc as plsc

EMBED_DIM = 128
BATCH = 16384
NUM_CORES = 2
NUM_SUBCORES = 16
NUM_WORKERS = NUM_CORES * NUM_SUBCORES
ROWS_PER_W = BATCH // NUM_WORKERS
CHUNK = 128
NCHUNK = ROWS_PER_W // CHUNK
RING = 3
LANES = 16


def _build_tables_body(res_ref, inc_ref, typ_ref, wrk_ref, w_ref, b_ref,
                       t12_ref, t34_ref):
    w = w_ref[...]
    t1 = jnp.dot(res_ref[...], w[0:8, :], preferred_element_type=jnp.float32)
    t2 = jnp.dot(inc_ref[...], w[8:24, :], preferred_element_type=jnp.float32)
    t3 = jnp.dot(typ_ref[...], w[24:32, :], preferred_element_type=jnp.float32)
    t4 = jnp.dot(wrk_ref[...], w[32:40, :], preferred_element_type=jnp.float32)
    bias = b_ref[...]
    for i in range(20):
        t12_ref[pl.ds(i * 20, 20), :] = t1[i:i + 1, :] + t2
    t4b = t4 + bias
    for i in range(10):
        t34_ref[pl.ds(i * 10, 10), :] = t3[i:i + 1, :] + t4b


def _build_tables(emb_res, emb_inc, emb_typ, emb_wrk, W, b):
    return pl.pallas_call(
        _build_tables_body,
        out_shape=(
            jax.ShapeDtypeStruct((400, EMBED_DIM), jnp.float32),
            jax.ShapeDtypeStruct((100, EMBED_DIM), jnp.float32),
        ),
    )(emb_res, emb_inc, emb_typ, emb_wrk, W, b.reshape(1, EMBED_DIM))


def _sc_lookup(xt, t12, t34):
    mesh = plsc.VectorSubcoreMesh(core_axis_name="c", subcore_axis_name="s")

    @functools.partial(
        pl.kernel,
        mesh=mesh,
        out_type=jax.ShapeDtypeStruct((BATCH, EMBED_DIM), jnp.float32),
        scratch_types=[
            pltpu.VMEM((ROWS_PER_W,), jnp.int32),
            pltpu.VMEM((ROWS_PER_W,), jnp.int32),
            pltpu.VMEM((ROWS_PER_W,), jnp.int32),
            pltpu.VMEM((ROWS_PER_W,), jnp.int32),
            pltpu.VMEM((ROWS_PER_W,), jnp.int32),
            pltpu.VMEM((ROWS_PER_W,), jnp.int32),
        ] + [pltpu.VMEM((CHUNK, EMBED_DIM), jnp.float32)
             for _ in range(RING)]
          + [pltpu.VMEM((CHUNK, EMBED_DIM), jnp.float32)
             for _ in range(RING)]
          + [
            pltpu.VMEM_SHARED((400, EMBED_DIM), jnp.float32),
            pltpu.VMEM_SHARED((100, EMBED_DIM), jnp.float32),
            pltpu.SemaphoreType.DMA,
        ] + [pltpu.SemaphoreType.DMA for _ in range(RING)]
          + [pltpu.SemaphoreType.DMA for _ in range(RING)],
    )
    def k(xth, t12h, t34h, outh,
          x0v, x1v, x2v, x3v, i12v, i34v, *rest):
        b12s = list(rest[0:RING])
        b34s = list(rest[RING:2 * RING])
        t12s, t34s, sx = rest[2 * RING], rest[2 * RING + 1], rest[2 * RING + 2]
        sg = list(rest[2 * RING + 3:2 * RING + 3 + RING])
        so = list(rest[2 * RING + 3 + RING:2 * RING + 3 + 2 * RING])
        sid = lax.axis_index("s")
        wid = sid * NUM_CORES + lax.axis_index("c")
        base = wid * ROWS_PER_W

        xcp = [pltpu.async_copy(xth.at[f, pl.ds(base, ROWS_PER_W)], xv, sx)
               for f, xv in enumerate([x0v, x1v, x2v, x3v])]

        @pl.when(sid == 0)
        def _stage_tables():
            pltpu.sync_copy(t12h, t12s)
            pltpu.sync_copy(t34h, t34s)

        for c in xcp:
            c.wait()
        for r in range(ROWS_PER_W // LANES):
            sl = pl.ds(r * LANES, LANES)
            i12v[sl] = x0v[sl] * 20 + x1v[sl]
            i34v[sl] = x2v[sl] * 10 + x3v[sl]

        plsc.subcore_barrier()

        def issue(c):
            s = c % RING
            isl = pl.ds(c * CHUNK, CHUNK)
            return (pltpu.async_copy(t12s.at[i12v.at[isl]], b12s[s], sg[s]),
                    pltpu.async_copy(t34s.at[i34v.at[isl]], b34s[s], sg[s]))

        UNROLL = 4
        LOOKAHEAD = RING - 1
        gcp = [None] * NCHUNK
        ocp = [None] * NCHUNK
        for c0 in range(min(LOOKAHEAD, NCHUNK)):
            gcp[c0] = issue(c0)
        for c in range(NCHUNK):
            s = c % RING
            n = c + LOOKAHEAD
            if n < NCHUNK:
                if n - RING >= 0:
                    ocp[n - RING].wait()
                gcp[n] = issue(n)
            gcp[c][0].wait()
            gcp[c][1].wait()
            b12, b34 = b12s[s], b34s[s]

            def add_body(r, carry):
                for u in range(UNROLL):
                    for jj in range(EMBED_DIM // LANES):
                        sl = pl.ds(jj * LANES, LANES)
                        plsc.addupdate(b12.at[r * UNROLL + u, sl],
                                       b34[r * UNROLL + u, sl])
                return carry

            lax.fori_loop(0, CHUNK // UNROLL, add_body, 0)
            ocp[c] = pltpu.async_copy(
                b12, outh.at[pl.ds(base + c * CHUNK, CHUNK)], so[s])
        for c in range(max(0, NCHUNK - RING), NCHUNK):
            ocp[c].wait()

    return k(xt, t12, t34)


def kernel(x, emb_res, emb_inc, emb_typ, emb_wrk, W, b):
    t12, t34 = _build_tables(emb_res, emb_inc, emb_typ, emb_wrk, W, b)
    out = _sc_lookup(x.astype(jnp.int32).T, t12, t34)
    return out[:, None, :]
